# baseline (speedup 1.0000x reference)
import functools
import math

import jax
import jax.numpy as jnp
from jax import lax
from jax.experimental import pallas as pl
from jax.experimental.pallas import tpu as pltpu
from jax.experimental.pallas import tpu_sc as plsc

NMOD = 3
R = 16
NFF = 128
B = 4096
NV = 1000000

_NC = 2
_NS = 16
_NW = _NC * _NS
_BPW = B // _NW
_CHUNK = 16
_NBUF = 2


def _gather_body(ib, u0, u1, u2, out, idx_v, col0, col1, col2, tile_v, sem):
    wid = lax.axis_index("s") * _NC + lax.axis_index("c")
    base = wid * _BPW
    pltpu.sync_copy(ib.at[:, pl.ds(base, _BPW)], idx_v)
    tabs = (u0, u1, u2)
    cols = (col0, col1, col2)
    row_iota = lax.iota(jnp.int32, R)
    nch = _BPW // _CHUNK

    def fire(m, c, buf):
        vec = idx_v[m, pl.ds(c * _CHUNK, _CHUNK)]
        for k in range(_CHUNK):
            col_base = pl.multiple_of((vec[k] >> 7) << 7, 128)
            pltpu.async_copy(tabs[m].at[:, pl.ds(col_base, 128)],
                             tile_v.at[buf, k], sem)
        return vec

    def drain(m, c, buf, vec):
        lanes = vec & 127
        for k in range(_CHUNK):
            pltpu.make_async_copy(tabs[m].at[:, pl.ds(0, 128)],
                                  tile_v.at[buf, k], sem).wait()
            lane = jnp.full((R,), lanes[k], jnp.int32)
            v = plsc.load_gather(tile_v.at[buf, k], [row_iota, lane])
            plsc.store_scatter(
                cols[m], [row_iota, jnp.full((R,), c * _CHUNK + k, jnp.int32)],
                v)

    for m in range(NMOD):
        vec0 = fire(m, 0, 0)

        @pl.loop(0, nch - 2, step=2, init_carry=vec0)
        def _pair(c, veca, m=m):
            vecb = fire(m, c + 1, 1)
            drain(m, c, 0, veca)
            vecc = fire(m, c + 2, 0)
            drain(m, c + 1, 1, vecb)
            return vecc

        veca = _pair
        vecb = fire(m, nch - 1, 1)
        drain(m, nch - 2, 0, veca)
        drain(m, nch - 1, 1, vecb)
    pltpu.sync_copy(col0, out.at[0, :, pl.ds(base, _BPW)])
    pltpu.sync_copy(col1, out.at[1, :, pl.ds(base, _BPW)])
    pltpu.sync_copy(col2, out.at[2, :, pl.ds(base, _BPW)])


@functools.cache
def _sc_gather():
    return pl.kernel(
        _gather_body,
        out_type=jax.ShapeDtypeStruct((NMOD, R, B), jnp.float32),
        mesh=plsc.VectorSubcoreMesh(core_axis_name="c", subcore_axis_name="s",
                                    num_cores=_NC, num_subcores=_NS),
        scratch_types=[
            pltpu.VMEM((NMOD, _BPW), jnp.int32),
            pltpu.VMEM((R, _BPW), jnp.float32),
            pltpu.VMEM((R, _BPW), jnp.float32),
            pltpu.VMEM((R, _BPW), jnp.float32),
            pltpu.VMEM((_NBUF, _CHUNK, R, 128), jnp.float32),
            pltpu.SemaphoreType.DMA,
        ],
        compiler_params=pltpu.CompilerParams(use_tc_tiling_on_sc=True,
                                             needs_layout_passes=False),
    )


def _mlp_body(g_ref, t_ref, wff_ref, wout_ref, y_ref):
    w = wff_ref[...]
    dn = (((0,), (0,)), ((), ()))
    proj = jnp.dot(t_ref[...], w[NMOD * R:NMOD * R + 1],
                   preferred_element_type=jnp.float32)
    for m in range(NMOD):
        proj = proj + lax.dot_general(g_ref[m], w[m * R:(m + 1) * R], dn,
                                      preferred_element_type=jnp.float32)
    scale = 1.0 / math.sqrt(NFF)
    wo = wout_ref[...]
    y = jnp.sum(jnp.sin(proj) * wo[:, 0:NFF]
                + jnp.cos(proj) * wo[:, NFF:2 * NFF], axis=1)
    y_ref[...] = y * scale


_mlp = pl.pallas_call(
    _mlp_body,
    out_shape=jax.ShapeDtypeStruct((B,), jnp.float32),
)


def kernel(b_i_n, b_t_n, U0, U1, U2, W_ff, w_out):
    g = _sc_gather()(b_i_n.astype(jnp.int32).T, U0.T, U1.T, U2.T)
    y = _mlp(g, b_t_n.reshape(B, 1), W_ff, w_out.T)
    return y.reshape(B, 1)


# speedup vs baseline: 1.0282x; 1.0282x over previous
import functools
import math

import jax
import jax.numpy as jnp
from jax import lax
from jax.experimental import pallas as pl
from jax.experimental.pallas import tpu as pltpu
from jax.experimental.pallas import tpu_sc as plsc

NMOD = 3
R = 16
NFF = 128
B = 4096
NV = 1000000

_NC = 2
_NS = 16
_NW = _NC * _NS
_BPW = B // _NW
_CHUNK = 16
_NBUF = 2


def _gather_body(ib, u0, u1, u2, out, idx_v, col0, col1, col2, tile_v, sem):
    wid = lax.axis_index("s") * _NC + lax.axis_index("c")
    base = wid * _BPW
    pltpu.sync_copy(ib.at[:, pl.ds(base, _BPW)], idx_v)
    tabs = (u0, u1, u2)
    cols = (col0, col1, col2)
    row_iota = lax.iota(jnp.int32, R)
    nch = _BPW // _CHUNK

    def fire(m, c, buf):
        vec = idx_v[m, pl.ds(c * _CHUNK, _CHUNK)]
        for k in range(_CHUNK):
            col_base = pl.multiple_of((vec[k] >> 7) << 7, 128)
            pltpu.async_copy(tabs[m].at[:, pl.ds(col_base, 128)],
                             tile_v.at[buf, k], sem)
        return vec

    def drain(m, c, buf, vec):
        lanes = vec & 127
        for k in range(_CHUNK):
            pltpu.make_async_copy(tabs[m].at[:, pl.ds(0, 128)],
                                  tile_v.at[buf, k], sem).wait()
            lane = jnp.full((R,), lanes[k], jnp.int32)
            v = plsc.load_gather(tile_v.at[buf, k], [row_iota, lane])
            plsc.store_scatter(
                cols[m], [row_iota, jnp.full((R,), c * _CHUNK + k, jnp.int32)],
                v)

    vec0 = fire(0, 0, 0)
    for m in range(NMOD):

        @pl.loop(0, nch - 2, step=2, init_carry=vec0)
        def _pair(c, veca, m=m):
            vecb = fire(m, c + 1, 1)
            drain(m, c, 0, veca)
            vecc = fire(m, c + 2, 0)
            drain(m, c + 1, 1, vecb)
            return vecc

        veca = _pair
        vecb = fire(m, nch - 1, 1)
        drain(m, nch - 2, 0, veca)
        if m + 1 < NMOD:
            vec0 = fire(m + 1, 0, 0)
        drain(m, nch - 1, 1, vecb)
    pltpu.sync_copy(col0, out.at[0, :, pl.ds(base, _BPW)])
    pltpu.sync_copy(col1, out.at[1, :, pl.ds(base, _BPW)])
    pltpu.sync_copy(col2, out.at[2, :, pl.ds(base, _BPW)])


@functools.cache
def _sc_gather():
    return pl.kernel(
        _gather_body,
        out_type=jax.ShapeDtypeStruct((NMOD, R, B), jnp.float32),
        mesh=plsc.VectorSubcoreMesh(core_axis_name="c", subcore_axis_name="s",
                                    num_cores=_NC, num_subcores=_NS),
        scratch_types=[
            pltpu.VMEM((NMOD, _BPW), jnp.int32),
            pltpu.VMEM((R, _BPW), jnp.float32),
            pltpu.VMEM((R, _BPW), jnp.float32),
            pltpu.VMEM((R, _BPW), jnp.float32),
            pltpu.VMEM((_NBUF, _CHUNK, R, 128), jnp.float32),
            pltpu.SemaphoreType.DMA,
        ],
        compiler_params=pltpu.CompilerParams(use_tc_tiling_on_sc=True,
                                             needs_layout_passes=False),
    )


def _mlp_body(g_ref, t_ref, wff_ref, wout_ref, y_ref):
    w = wff_ref[...]
    dn = (((0,), (0,)), ((), ()))
    proj = jnp.dot(t_ref[...], w[NMOD * R:NMOD * R + 1],
                   preferred_element_type=jnp.float32)
    for m in range(NMOD):
        proj = proj + lax.dot_general(g_ref[m], w[m * R:(m + 1) * R], dn,
                                      preferred_element_type=jnp.float32)
    scale = 1.0 / math.sqrt(NFF)
    wo = wout_ref[...]
    y = jnp.sum(jnp.sin(proj) * wo[:, 0:NFF]
                + jnp.cos(proj) * wo[:, NFF:2 * NFF], axis=1)
    y_ref[...] = y * scale


_mlp = pl.pallas_call(
    _mlp_body,
    out_shape=jax.ShapeDtypeStruct((B,), jnp.float32),
)


def kernel(b_i_n, b_t_n, U0, U1, U2, W_ff, w_out):
    g = _sc_gather()(b_i_n.astype(jnp.int32).T, U0.T, U1.T, U2.T)
    y = _mlp(g, b_t_n.reshape(B, 1), W_ff, w_out.T)
    return y.reshape(B, 1)
